# Initial kernel scaffold; baseline (speedup 1.0000x reference)
import jax
import jax.numpy as jnp
from jax.experimental import pallas as pl


def kernel(x, edge_index, edge_weight, s1_tc1_w, s1_tc1_b, s1_cheb_w, s1_cheb_b, s1_tc2_w, s1_tc2_b, s1_bn_g, s1_bn_b, s2_tc1_w, s2_tc1_b, s2_cheb_w, s2_cheb_b, s2_tc2_w, s2_tc2_b, s2_bn_g, s2_bn_b, w_final, b_final, w_lin, b_lin):
    raise NotImplementedError("write your pallas kernel here")


# speedup vs baseline: 22.3124x; 22.3124x over previous
import functools

import jax
import jax.numpy as jnp
from jax import lax
from jax.experimental import pallas as pl
from jax.experimental.pallas import tpu as pltpu
from jax.experimental.pallas import tpu_sc as plsc

N = 10000
NP = 10240
NW = NP // 2
NW2 = NP * NW
E = 160000
EP = 163840
CHUNK = 128
B = 2
T = 12


def _vmesh(num_cores):
  return plsc.VectorSubcoreMesh(
      core_axis_name="c", subcore_axis_name="s",
      num_cores=num_cores, num_subcores=16)


def _zero_vmem(buf, n):
  def step(j, _):
    buf[pl.ds(j * 16, 16)] = jnp.zeros((16,), buf.dtype)
    return _
  lax.fori_loop(0, n // 16, step, None)


def _sc_deg_and_owner(row_p, col_p, ew_p):
  EPT = EP // 32
  NCH = EPT // CHUNK

  @functools.partial(
      pl.kernel,
      out_type=(jax.ShapeDtypeStruct((NW2,), jnp.int32),
                jax.ShapeDtypeStruct((2, NP), jnp.float32)),
      mesh=_vmesh(2),
      scratch_types=[
          pltpu.VMEM((CHUNK,), jnp.int32),
          pltpu.VMEM((CHUNK,), jnp.int32),
          pltpu.VMEM((CHUNK,), jnp.float32),
          pltpu.VMEM((CHUNK,), jnp.int32),
          pltpu.VMEM((CHUNK,), jnp.int32),
          pltpu.VMEM((CHUNK,), jnp.float32),
          pltpu.VMEM((1024,), jnp.float32),
          pltpu.VMEM_SHARED((NP,), jnp.float32),
          pltpu.SemaphoreType.DMA,
      ],
  )
  def k(row_h, col_h, ew_h, d_h, degp_h,
        rbuf, cbuf, wbuf, keybuf, eidbuf, wzbuf, zchunk, degacc, sem):
    c = lax.axis_index("c")
    s = lax.axis_index("s")
    wid = c * 16 + s

    @pl.when(s == 0)
    def _():
      _zero_vmem(zchunk, 1024)
      def zs(i, _):
        pltpu.sync_copy(zchunk, degacc.at[pl.ds(i * 1024, 1024)])
        return _
      lax.fori_loop(0, NP // 1024, zs, None)

    plsc.subcore_barrier()
    base = wid * EPT

    def chunk_step(i, _):
      off = base + i * CHUNK
      pltpu.sync_copy(row_h.at[pl.ds(off, CHUNK)], rbuf)
      pltpu.sync_copy(col_h.at[pl.ds(off, CHUNK)], cbuf)
      pltpu.sync_copy(ew_h.at[pl.ds(off, CHUNK)], wbuf)
      for j in range(CHUNK // 16):
        sl = pl.ds(j * 16, 16)
        r = rbuf[sl]
        co = cbuf[sl]
        w = wbuf[sl]
        keybuf[sl] = r * NW + lax.shift_right_logical(co, 1)
        eidbuf[sl] = lax.iota(jnp.int32, 16) + (off + j * 16)
        wzbuf[sl] = jnp.where(r == co, 0.0, w)
      pltpu.async_copy(eidbuf, d_h.at[keybuf], sem).wait()
      pltpu.sync_copy(wzbuf, degacc.at[rbuf], add=True)
      return _

    lax.fori_loop(0, NCH, chunk_step, None)
    plsc.subcore_barrier()

    @pl.when(s == 0)
    def _():
      pltpu.sync_copy(degacc, degp_h.at[c])

  return k(row_p, col_p, ew_p)


def _tc_dinv(degp):
  def body(degp_ref, out_ref):
    sm = degp_ref[0:1, :] + degp_ref[1:2, :]
    out_ref[...] = jnp.where(sm > 0, lax.rsqrt(sm), 0.0)
  return pl.pallas_call(
      body, out_shape=jax.ShapeDtypeStruct((1, NP), jnp.float32))(degp)


def _sc_lw_sums(row_p, col_p, ew_p, dinv, d_tab):
  EPT = EP // 32
  NCH = EPT // CHUNK

  @functools.partial(
      pl.kernel,
      out_type=(jax.ShapeDtypeStruct((EP,), jnp.int32),
                jax.ShapeDtypeStruct((2, 2, EP), jnp.float32)),
      mesh=_vmesh(2),
      scratch_types=[
          pltpu.VMEM((CHUNK,), jnp.int32),
          pltpu.VMEM((CHUNK,), jnp.int32),
          pltpu.VMEM((CHUNK,), jnp.float32),
          pltpu.VMEM((CHUNK,), jnp.int32),
          pltpu.VMEM((CHUNK,), jnp.int32),
          pltpu.VMEM((CHUNK,), jnp.float32),
          pltpu.VMEM((CHUNK,), jnp.float32),
          pltpu.VMEM((CHUNK,), jnp.float32),
          pltpu.VMEM((CHUNK,), jnp.float32),
          pltpu.VMEM((16384,), jnp.float32),
          pltpu.VMEM_SHARED((EP,), jnp.float32),
          pltpu.VMEM_SHARED((EP,), jnp.float32),
          pltpu.SemaphoreType.DMA,
      ],
  )
  def k(row_h, col_h, ew_h, dinv_h, d_h, win_h, lwp_h,
        rbuf, cbuf, wbuf, keybuf, winbuf, evbuf, odbuf, drbuf, dcbuf,
        zchunk, lws_ev, lws_od, sem):
    c = lax.axis_index("c")
    s = lax.axis_index("s")
    wid = c * 16 + s

    @pl.when(s == 0)
    def _():
      _zero_vmem(zchunk, 16384)
      def zs(i, _):
        pltpu.sync_copy(zchunk, lws_ev.at[pl.ds(i * 16384, 16384)])
        pltpu.sync_copy(zchunk, lws_od.at[pl.ds(i * 16384, 16384)])
        return _
      lax.fori_loop(0, EP // 16384, zs, None)

    plsc.subcore_barrier()
    base = wid * EPT

    def chunk_step(i, _):
      off = base + i * CHUNK
      pltpu.sync_copy(row_h.at[pl.ds(off, CHUNK)], rbuf)
      pltpu.sync_copy(col_h.at[pl.ds(off, CHUNK)], cbuf)
      pltpu.sync_copy(ew_h.at[pl.ds(off, CHUNK)], wbuf)
      for j in range(CHUNK // 16):
        sl = pl.ds(j * 16, 16)
        keybuf[sl] = rbuf[sl] * NW + lax.shift_right_logical(cbuf[sl], 1)
      pltpu.async_copy(d_h.at[keybuf], winbuf, sem).wait()
      pltpu.async_copy(dinv_h.at[rbuf], drbuf, sem).wait()
      pltpu.async_copy(dinv_h.at[cbuf], dcbuf, sem).wait()
      pltpu.sync_copy(winbuf, win_h.at[pl.ds(off, CHUNK)])
      for j in range(CHUNK // 16):
        sl = pl.ds(j * 16, 16)
        r = rbuf[sl]
        co = cbuf[sl]
        lw = jnp.where(r == co, 0.0, -(drbuf[sl] * wbuf[sl] * dcbuf[sl]))
        par = lax.bitwise_and(co, 1)
        evbuf[sl] = jnp.where(par == 0, lw, 0.0)
        odbuf[sl] = jnp.where(par == 1, lw, 0.0)
      pltpu.sync_copy(evbuf, lws_ev.at[winbuf], add=True)
      pltpu.sync_copy(odbuf, lws_od.at[winbuf], add=True)
      return _

    lax.fori_loop(0, NCH, chunk_step, None)
    plsc.subcore_barrier()

    @pl.when(s == 0)
    def _():
      pltpu.sync_copy(lws_ev, lwp_h.at[c, 0])
      pltpu.sync_copy(lws_od, lwp_h.at[c, 1])

  return k(row_p, col_p, ew_p, dinv, d_tab)


def _tc_pack(row2, col2, win2, lwp):
  rows = EP // CHUNK

  def body(r_ref, c_ref, w_ref, lwp_ref, key_ref, word_ref):
    def rne16(v):
      b = lax.bitcast_convert_type(v, jnp.int32)
      rnd = b + 0x7FFF + lax.bitwise_and(lax.shift_right_logical(b, 16), 1)
      return lax.shift_right_logical(rnd, 16)

    ev = rne16(lwp_ref[0, 0] + lwp_ref[1, 0])
    od = rne16(lwp_ref[0, 1] + lwp_ref[1, 1])
    word_ref[...] = lax.bitwise_or(ev, lax.shift_left(od, 16))
    eid = (lax.broadcasted_iota(jnp.int32, (rows, CHUNK), 0) * CHUNK
           + lax.broadcasted_iota(jnp.int32, (rows, CHUNK), 1))
    m = w_ref[...] == eid
    key = r_ref[...] * NW + lax.shift_right_logical(c_ref[...], 1)
    key_ref[...] = jnp.where(m, key, N * NW + eid)

  return pl.pallas_call(
      body,
      out_shape=[jax.ShapeDtypeStruct((rows, CHUNK), jnp.int32)] * 2,
  )(row2, col2, win2, lwp.reshape(2, 2, rows, CHUNK))


def _sc_build_l(keys, words):
  EPT = EP // 16
  NCH = EPT // CHUNK
  STRIPE = NW2 // 16
  ZC = 65536

  @functools.partial(
      pl.kernel,
      out_type=jax.ShapeDtypeStruct((NW2,), jnp.int32),
      mesh=_vmesh(1),
      scratch_types=[
          pltpu.VMEM((CHUNK,), jnp.int32),
          pltpu.VMEM((CHUNK,), jnp.int32),
          pltpu.VMEM((ZC,), jnp.int32),
          pltpu.SemaphoreType.DMA,
      ],
  )
  def k(key_h, word_h, l_h, keybuf, wordbuf, zchunk, sem):
    s = lax.axis_index("s")
    _zero_vmem(zchunk, ZC)

    def zs(i, _):
      pltpu.sync_copy(zchunk, l_h.at[pl.ds(s * STRIPE + i * ZC, ZC)])
      return _
    lax.fori_loop(0, STRIPE // ZC, zs, None)
    plsc.subcore_barrier()
    base = s * EPT

    def chunk_step(i, _):
      off = base + i * CHUNK
      pltpu.sync_copy(key_h.at[pl.ds(off, CHUNK)], keybuf)
      pltpu.sync_copy(word_h.at[pl.ds(off, CHUNK)], wordbuf)
      pltpu.async_copy(wordbuf, l_h.at[keybuf], sem).wait()
      return _

    lax.fori_loop(0, NCH, chunk_step, None)

  return k(keys, words)


def _split(v):
  hi = v.astype(jnp.bfloat16)
  lo = (v - hi.astype(jnp.float32)).astype(jnp.bfloat16)
  return hi, lo


def _mm3(ah, al, whl):
  wh, wl = whl
  return (jnp.dot(ah, wh, preferred_element_type=jnp.float32)
          + jnp.dot(ah, wl, preferred_element_type=jnp.float32)
          + jnp.dot(al, wh, preferred_element_type=jnp.float32))


def _spmm(l16, xhi, f):
  BM, BK = 2560, 1024
  nk = NP // BK

  def body(l_ref, x_ref, yhi_ref, ylo_ref, acc_ref):
    k = pl.program_id(1)

    @pl.when(k == 0)
    def _():
      acc_ref[...] = jnp.zeros_like(acc_ref)

    acc_ref[...] += jnp.dot(l_ref[...], x_ref[...],
                            preferred_element_type=jnp.float32)

    @pl.when(k == nk - 1)
    def _():
      hi, lo = _split(acc_ref[...])
      yhi_ref[...] = hi
      ylo_ref[...] = lo

  return pl.pallas_call(
      body,
      grid=(NP // BM, nk),
      in_specs=[
          pl.BlockSpec((BM, BK), lambda i, k: (i, k)),
          pl.BlockSpec((BK, f), lambda i, k: (k, 0)),
      ],
      out_specs=[pl.BlockSpec((BM, f), lambda i, k: (i, 0))] * 2,
      out_shape=[jax.ShapeDtypeStruct((NP, f), jnp.bfloat16)] * 2,
      scratch_shapes=[pltpu.VMEM((BM, f), jnp.float32)],
      compiler_params=pltpu.CompilerParams(
          dimension_semantics=("parallel", "arbitrary")),
  )(l16, xhi)


def _gate(ah, al, wp, wq, wr, pb, qb, rb):
  p = _mm3(ah, al, wp) + pb
  q = _mm3(ah, al, wq) + qb
  r = _mm3(ah, al, wr) + rb
  return jax.nn.relu(p * jax.nn.sigmoid(q) + r)


def _tc1_call(xnm, wp, wq, wr, pb, qb, rb):
  BM = 2560
  f_in, f_out = xnm.shape[1], pb.shape[1]

  def body(x_ref, wph, wpl, wqh, wql, wrh, wrl, pb_r, qb_r, rb_r,
           hhi_ref, hlo_ref):
    ah, al = _split(x_ref[...])
    h = _gate(ah, al, (wph[...], wpl[...]), (wqh[...], wql[...]),
              (wrh[...], wrl[...]), pb_r[...], qb_r[...], rb_r[...])
    hi, lo = _split(h)
    hhi_ref[...] = hi
    hlo_ref[...] = lo

  full = lambda shape: pl.BlockSpec(shape, lambda i: (0, 0))
  return pl.pallas_call(
      body,
      grid=(NP // BM,),
      in_specs=[pl.BlockSpec((BM, f_in), lambda i: (i, 0))]
      + [full((f_in, f_out))] * 6 + [full((1, f_out))] * 3,
      out_specs=[pl.BlockSpec((BM, f_out), lambda i: (i, 0))] * 2,
      out_shape=[jax.ShapeDtypeStruct((NP, f_out), jnp.bfloat16)] * 2,
  )(xnm, wp[0], wp[1], wq[0], wq[1], wr[0], wr[1], pb, qb, rb)


def _epilogue_call(yhi, ylo, t1hi, t1lo, h0hi, h0lo, cheb_w, tc2_w, bn_gb,
                   tail_w, f_sizes, out_f32):
  BM = 1280
  f, f2, f3 = f_sizes
  ca, cb, cc, cbias = cheb_w
  wp, wq, wr, pb, qb, rb = tc2_w
  g_col, b_col = bn_gb
  twp, twq, twr, tpb, tqb, trb = tail_w

  def body(yhi_r, ylo_r, t1hi_r, t1lo_r, h0hi_r, h0lo_r,
           cah, cal, cbh, cbl, cch, ccl, cbias_r,
           wph, wpl, wqh, wql, wrh, wrl, pb_r, qb_r, rb_r,
           g_r, b_r,
           twph, twpl, twqh, twql, twrh, twrl, tpb_r, tqb_r, trb_r,
           out_hi_ref, out_lo_ref):
    hc = jax.nn.relu(
        _mm3(h0hi_r[...], h0lo_r[...], (cah[...], cal[...]))
        + _mm3(t1hi_r[...], t1lo_r[...], (cbh[...], cbl[...]))
        + _mm3(yhi_r[...], ylo_r[...], (cch[...], ccl[...]))
        + cbias_r[...])
    hh, hl = _split(hc)
    gt = _gate(hh, hl, (wph[...], wpl[...]), (wqh[...], wql[...]),
               (wrh[...], wrl[...]), pb_r[...], qb_r[...], rb_r[...])
    mu = jnp.mean(gt, axis=1, keepdims=True)
    xc = gt - mu
    var = jnp.mean(xc * xc, axis=1, keepdims=True)
    xn = xc * lax.rsqrt(var + 1e-5) * g_r[...] + b_r[...]
    xh, xl = _split(xn)
    if out_f32:
      out_hi_ref[...] = (_mm3(xh, xl, (twph[...], twpl[...]))
                         + tpb_r[...])
      out_lo_ref[...] = jnp.zeros(out_lo_ref.shape, out_lo_ref.dtype)
    else:
      h2 = _gate(xh, xl, (twph[...], twpl[...]), (twqh[...], twql[...]),
                 (twrh[...], twrl[...]), tpb_r[...], tqb_r[...], trb_r[...])
      hi, lo = _split(h2)
      out_hi_ref[...] = hi
      out_lo_ref[...] = lo

  fo = tpb.shape[1]
  odt = jnp.float32 if out_f32 else jnp.bfloat16
  blk = lambda w: pl.BlockSpec(w.shape, lambda i: tuple(0 for _ in w.shape))
  row = lambda ff: pl.BlockSpec((BM, ff), lambda i: (i, 0))
  ins = [yhi, ylo, t1hi, t1lo, h0hi, h0lo,
         ca[0], ca[1], cb[0], cb[1], cc[0], cc[1], cbias,
         wp[0], wp[1], wq[0], wq[1], wr[0], wr[1], pb, qb, rb,
         g_col, b_col,
         twp[0], twp[1], twq[0], twq[1], twr[0], twr[1], tpb, tqb, trb]
  in_specs = ([row(f)] * 6
              + [blk(a) for a in ins[6:22]]
              + [pl.BlockSpec((BM, 1), lambda i: (i, 0))] * 2
              + [blk(a) for a in ins[24:]])
  return pl.pallas_call(
      body,
      grid=(NP // BM,),
      in_specs=in_specs,
      out_specs=[pl.BlockSpec((BM, fo), lambda i: (i, 0))] * 2,
      out_shape=[jax.ShapeDtypeStruct((NP, fo), odt)] * 2,
  )(*ins)


def _tconv_mats(w, b, t_in, i_ch, o_ch):
  ks = w.shape[-1]
  t_out = t_in - ks + 1
  mats, biases = [], []
  for gi in range(3):
    m1 = sum(
        jnp.einsum("ab,io->aibo",
                   jnp.eye(t_in, t_out, -kk, dtype=jnp.float32),
                   w[gi, :, :, 0, kk].T)
        for kk in range(ks))
    m = jnp.einsum("xy,tiso->xtiyso", jnp.eye(B, dtype=jnp.float32), m1)
    mats.append(_split(m.reshape(B * t_in * i_ch, B * t_out * o_ch)))
    biases.append(jnp.broadcast_to(b[gi], (B, t_out, o_ch)).reshape(1, -1))
  return mats, biases


def _cheb_mats(chw, chb, bt):
  eye = jnp.eye(bt, dtype=jnp.float32)
  ca = _split(jnp.kron(eye, (chw[0] - chw[2]).T))
  cb = _split(jnp.kron(eye, chw[1].T))
  cc = _split(jnp.kron(eye, 2.0 * chw[2].T))
  cbias = jnp.broadcast_to(chb, (bt, chw.shape[1])).reshape(1, -1)
  return ca, cb, cc, cbias


def _pad_rows(a, n_to):
  return jnp.pad(a, ((0, n_to - a.shape[0]),) + ((0, 0),) * (a.ndim - 1))


def kernel(x, edge_index, edge_weight, s1_tc1_w, s1_tc1_b, s1_cheb_w,
           s1_cheb_b, s1_tc2_w, s1_tc2_b, s1_bn_g, s1_bn_b, s2_tc1_w,
           s2_tc1_b, s2_cheb_w, s2_cheb_b, s2_tc2_w, s2_tc2_b, s2_bn_g,
           s2_bn_b, w_final, b_final, w_lin, b_lin):
  row = jnp.pad(edge_index[0].astype(jnp.int32), (0, EP - E))
  col = jnp.pad(edge_index[1].astype(jnp.int32), (0, EP - E))
  ew = jnp.pad(edge_weight, (0, EP - E))

  d_tab, degp = _sc_deg_and_owner(row, col, ew)
  dinv = _tc_dinv(degp).reshape(NP)
  win, lwp = _sc_lw_sums(row, col, ew, dinv, d_tab)
  keys, words = _tc_pack(row.reshape(-1, CHUNK), col.reshape(-1, CHUNK),
                         win.reshape(-1, CHUNK), lwp)
  l_words = _sc_build_l(keys.reshape(EP), words.reshape(EP))
  l16 = lax.bitcast_convert_type(
      l_words.reshape(NP, NW), jnp.bfloat16).reshape(NP, NP)

  tc1m, tc1b = _tconv_mats(s1_tc1_w, s1_tc1_b, T, 1, 32)
  cheb1 = _cheb_mats(s1_cheb_w, s1_cheb_b, B * 10)
  tc2m1, tc2b1 = _tconv_mats(s1_tc2_w, s1_tc2_b, 10, 32, 32)
  tc1m2, tc1b2 = _tconv_mats(s2_tc1_w, s2_tc1_b, 8, 32, 32)
  cheb2 = _cheb_mats(s2_cheb_w, s2_cheb_b, B * 6)
  tc2m2, tc2b2 = _tconv_mats(s2_tc2_w, s2_tc2_b, 6, 32, 32)
  wh_mat = jnp.kron(jnp.eye(B * 4, dtype=jnp.float32),
                    w_final @ w_lin.T)
  bh = jnp.broadcast_to(b_final @ w_lin.T + b_lin, (B * 4, 2)).reshape(1, -1)
  wh_pair = _split(wh_mat)
  g1 = _pad_rows(s1_bn_g.reshape(N, 1), NP)
  b1 = _pad_rows(s1_bn_b.reshape(N, 1), NP)
  g2 = _pad_rows(s2_bn_g.reshape(N, 1), NP)
  b2 = _pad_rows(s2_bn_b.reshape(N, 1), NP)

  xnm = _pad_rows(jnp.transpose(x[:, :, :, 0], (2, 0, 1)).reshape(N, B * T),
                  NP)
  h1hi, h1lo = _tc1_call(xnm, tc1m[0], tc1m[1], tc1m[2], *tc1b)
  t1hi, t1lo = _spmm(l16, h1hi, 640)
  yhi, ylo = _spmm(l16, t1hi, 640)
  h2hi, h2lo = _epilogue_call(
      yhi, ylo, t1hi, t1lo, h1hi, h1lo, cheb1,
      (tc2m1[0], tc2m1[1], tc2m1[2], *tc2b1), (g1, b1),
      (tc1m2[0], tc1m2[1], tc1m2[2], *tc1b2), (640, 512, 384), False)
  t2hi, t2lo = _spmm(l16, h2hi, 384)
  y2hi, y2lo = _spmm(l16, t2hi, 384)
  dummy = (jnp.zeros((1, 16), jnp.float32),) * 2
  ohi, _ = _epilogue_call(
      y2hi, y2lo, t2hi, t2lo, h2hi, h2lo, cheb2,
      (tc2m2[0], tc2m2[1], tc2m2[2], *tc2b2), (g2, b2),
      (wh_pair, (dummy[0].astype(jnp.bfloat16),) * 2,
       (dummy[0].astype(jnp.bfloat16),) * 2, bh, bh, bh),
      (384, 256, 16), True)
  return ohi[:N].reshape(N, B, 4, 2).transpose(1, 2, 0, 3)


# speedup vs baseline: 22.6915x; 1.0170x over previous
import functools

import jax
import jax.numpy as jnp
from jax import lax
from jax.experimental import pallas as pl
from jax.experimental.pallas import tpu as pltpu
from jax.experimental.pallas import tpu_sc as plsc

N = 10000
NP = 10240
NW = NP // 2
NW2 = NP * NW
E = 160000
EP = 163840
CHUNK = 128
B = 2
T = 12


def _vmesh(num_cores):
  return plsc.VectorSubcoreMesh(
      core_axis_name="c", subcore_axis_name="s",
      num_cores=num_cores, num_subcores=16)


def _zero_vmem(buf, n):
  def step(j, _):
    buf[pl.ds(j * 16, 16)] = jnp.zeros((16,), buf.dtype)
    return _
  lax.fori_loop(0, n // 16, step, None)


ROWS_E = EP // CHUNK
GRP = 8


def _sc_deg_and_owner(row_p, col_p, ew_p):
  RPT = ROWS_E // 32
  NO = RPT // GRP

  @functools.partial(
      pl.kernel,
      out_type=(jax.ShapeDtypeStruct((NW2,), jnp.int32),
                jax.ShapeDtypeStruct((2, NP), jnp.float32)),
      mesh=_vmesh(2),
      scratch_types=[
          pltpu.VMEM((GRP, CHUNK), jnp.int32),
          pltpu.VMEM((GRP, CHUNK), jnp.int32),
          pltpu.VMEM((GRP, CHUNK), jnp.float32),
          pltpu.VMEM((GRP, CHUNK), jnp.int32),
          pltpu.VMEM((GRP, CHUNK), jnp.int32),
          pltpu.VMEM((GRP, CHUNK), jnp.float32),
          pltpu.VMEM((1024,), jnp.float32),
          pltpu.VMEM_SHARED((NP,), jnp.float32),
          pltpu.SemaphoreType.DMA,
      ],
  )
  def k(row_h, col_h, ew_h, d_h, degp_h,
        row2, col2, w2, key2, eid2, wz2, zchunk, degacc, sem):
    c = lax.axis_index("c")
    s = lax.axis_index("s")
    wid = c * 16 + s

    @pl.when(s == 0)
    def _():
      _zero_vmem(zchunk, 1024)
      def zs(i, _):
        pltpu.sync_copy(zchunk, degacc.at[pl.ds(i * 1024, 1024)])
        return _
      lax.fori_loop(0, NP // 1024, zs, None)

    plsc.subcore_barrier()
    base = wid * RPT

    def outer(o, _):
      off = base + o * GRP
      pltpu.sync_copy(row_h.at[pl.ds(off, GRP)], row2)
      pltpu.sync_copy(col_h.at[pl.ds(off, GRP)], col2)
      pltpu.sync_copy(ew_h.at[pl.ds(off, GRP)], w2)
      for j in range(GRP):
        for v in range(CHUNK // 16):
          sl = pl.ds(v * 16, 16)
          r = row2[j, sl]
          co = col2[j, sl]
          key2[j, sl] = r * NW + lax.shift_right_logical(co, 1)
          eid2[j, sl] = (lax.iota(jnp.int32, 16)
                         + ((off + j) * CHUNK + v * 16))
          wz2[j, sl] = jnp.where(r == co, 0.0, w2[j, sl])
      cps = [pltpu.async_copy(eid2.at[j], d_h.at[key2.at[j]], sem)
             for j in range(GRP)]
      for cp in cps:
        cp.wait()
      for j in range(GRP):
        pltpu.sync_copy(wz2.at[j], degacc.at[row2.at[j]], add=True)
      return _

    lax.fori_loop(0, NO, outer, None)
    plsc.subcore_barrier()

    @pl.when(s == 0)
    def _():
      pltpu.sync_copy(degacc, degp_h.at[c])

  return k(row_p, col_p, ew_p)


def _tc_dinv(degp):
  def body(degp_ref, out_ref):
    sm = degp_ref[0:1, :] + degp_ref[1:2, :]
    out_ref[...] = jnp.where(sm > 0, lax.rsqrt(sm), 0.0)
  return pl.pallas_call(
      body, out_shape=jax.ShapeDtypeStruct((1, NP), jnp.float32))(degp)


def _sc_lw_sums(row_p, col_p, ew_p, dinv, d_tab):
  RPT = ROWS_E // 32
  NO = RPT // GRP

  @functools.partial(
      pl.kernel,
      out_type=(jax.ShapeDtypeStruct((ROWS_E, CHUNK), jnp.int32),
                jax.ShapeDtypeStruct((2, 2, EP), jnp.float32)),
      mesh=_vmesh(2),
      scratch_types=[
          pltpu.VMEM((GRP, CHUNK), jnp.int32),
          pltpu.VMEM((GRP, CHUNK), jnp.int32),
          pltpu.VMEM((GRP, CHUNK), jnp.float32),
          pltpu.VMEM((GRP, CHUNK), jnp.int32),
          pltpu.VMEM((GRP, CHUNK), jnp.int32),
          pltpu.VMEM((GRP, CHUNK), jnp.float32),
          pltpu.VMEM((GRP, CHUNK), jnp.float32),
          pltpu.VMEM((GRP, CHUNK), jnp.float32),
          pltpu.VMEM((GRP, CHUNK), jnp.float32),
          pltpu.VMEM((16384,), jnp.float32),
          pltpu.VMEM_SHARED((EP,), jnp.float32),
          pltpu.VMEM_SHARED((EP,), jnp.float32),
          pltpu.SemaphoreType.DMA,
      ],
  )
  def k(row_h, col_h, ew_h, dinv_h, d_h, win_h, lwp_h,
        row2, col2, w2, key2, win2, ev2, od2, dr2, dc2,
        zchunk, lws_ev, lws_od, sem):
    c = lax.axis_index("c")
    s = lax.axis_index("s")
    wid = c * 16 + s

    @pl.when(s == 0)
    def _():
      _zero_vmem(zchunk, 16384)
      def zs(i, _):
        pltpu.sync_copy(zchunk, lws_ev.at[pl.ds(i * 16384, 16384)])
        pltpu.sync_copy(zchunk, lws_od.at[pl.ds(i * 16384, 16384)])
        return _
      lax.fori_loop(0, EP // 16384, zs, None)

    plsc.subcore_barrier()
    base = wid * RPT

    def outer(o, _):
      off = base + o * GRP
      pltpu.sync_copy(row_h.at[pl.ds(off, GRP)], row2)
      pltpu.sync_copy(col_h.at[pl.ds(off, GRP)], col2)
      pltpu.sync_copy(ew_h.at[pl.ds(off, GRP)], w2)
      for j in range(GRP):
        for v in range(CHUNK // 16):
          sl = pl.ds(v * 16, 16)
          key2[j, sl] = (row2[j, sl] * NW
                         + lax.shift_right_logical(col2[j, sl], 1))
      cps = [pltpu.async_copy(d_h.at[key2.at[j]], win2.at[j], sem)
             for j in range(GRP)]
      cps += [pltpu.async_copy(dinv_h.at[row2.at[j]], dr2.at[j], sem)
              for j in range(GRP)]
      cps += [pltpu.async_copy(dinv_h.at[col2.at[j]], dc2.at[j], sem)
              for j in range(GRP)]
      for cp in cps:
        cp.wait()
      pltpu.sync_copy(win2, win_h.at[pl.ds(off, GRP)])
      for j in range(GRP):
        for v in range(CHUNK // 16):
          sl = pl.ds(v * 16, 16)
          r = row2[j, sl]
          co = col2[j, sl]
          lw = jnp.where(r == co, 0.0,
                         -(dr2[j, sl] * w2[j, sl] * dc2[j, sl]))
          par = lax.bitwise_and(co, 1)
          ev2[j, sl] = jnp.where(par == 0, lw, 0.0)
          od2[j, sl] = jnp.where(par == 1, lw, 0.0)
      for j in range(GRP):
        pltpu.sync_copy(ev2.at[j], lws_ev.at[win2.at[j]], add=True)
        pltpu.sync_copy(od2.at[j], lws_od.at[win2.at[j]], add=True)
      return _

    lax.fori_loop(0, NO, outer, None)
    plsc.subcore_barrier()

    @pl.when(s == 0)
    def _():
      pltpu.sync_copy(lws_ev, lwp_h.at[c, 0])
      pltpu.sync_copy(lws_od, lwp_h.at[c, 1])

  return k(row_p, col_p, ew_p, dinv, d_tab)


def _tc_pack(row2, col2, win2, lwp):
  rows = EP // CHUNK

  def body(r_ref, c_ref, w_ref, lwp_ref, key_ref, word_ref):
    def rne16(v):
      b = lax.bitcast_convert_type(v, jnp.int32)
      rnd = b + 0x7FFF + lax.bitwise_and(lax.shift_right_logical(b, 16), 1)
      return lax.shift_right_logical(rnd, 16)

    ev = rne16(lwp_ref[0, 0] + lwp_ref[1, 0])
    od = rne16(lwp_ref[0, 1] + lwp_ref[1, 1])
    word_ref[...] = lax.bitwise_or(ev, lax.shift_left(od, 16))
    eid = (lax.broadcasted_iota(jnp.int32, (rows, CHUNK), 0) * CHUNK
           + lax.broadcasted_iota(jnp.int32, (rows, CHUNK), 1))
    m = w_ref[...] == eid
    key = r_ref[...] * NW + lax.shift_right_logical(c_ref[...], 1)
    key_ref[...] = jnp.where(m, key, N * NW + eid)

  return pl.pallas_call(
      body,
      out_shape=[jax.ShapeDtypeStruct((rows, CHUNK), jnp.int32)] * 2,
  )(row2, col2, win2, lwp.reshape(2, 2, rows, CHUNK))


def _sc_build_l(keys, words):
  RPT = ROWS_E // 16
  NO = RPT // GRP
  STRIPE = NW2 // 16
  ZC = 65536

  @functools.partial(
      pl.kernel,
      out_type=jax.ShapeDtypeStruct((NW2,), jnp.int32),
      mesh=_vmesh(1),
      scratch_types=[
          pltpu.VMEM((GRP, CHUNK), jnp.int32),
          pltpu.VMEM((GRP, CHUNK), jnp.int32),
          pltpu.VMEM((ZC,), jnp.int32),
          pltpu.SemaphoreType.DMA,
      ],
  )
  def k(key_h, word_h, l_h, key2, word2, zchunk, sem):
    s = lax.axis_index("s")
    _zero_vmem(zchunk, ZC)

    def zs(i, _):
      pltpu.sync_copy(zchunk, l_h.at[pl.ds(s * STRIPE + i * ZC, ZC)])
      return _
    lax.fori_loop(0, STRIPE // ZC, zs, None)
    plsc.subcore_barrier()
    base = s * RPT

    def outer(o, _):
      off = base + o * GRP
      pltpu.sync_copy(key_h.at[pl.ds(off, GRP)], key2)
      pltpu.sync_copy(word_h.at[pl.ds(off, GRP)], word2)
      cps = [pltpu.async_copy(word2.at[j], l_h.at[key2.at[j]], sem)
             for j in range(GRP)]
      for cp in cps:
        cp.wait()
      return _

    lax.fori_loop(0, NO, outer, None)

  return k(keys, words)


def _split(v):
  hi = v.astype(jnp.bfloat16)
  lo = (v - hi.astype(jnp.float32)).astype(jnp.bfloat16)
  return hi, lo


def _mm3(ah, al, whl):
  wh, wl = whl
  return (jnp.dot(ah, wh, preferred_element_type=jnp.float32)
          + jnp.dot(ah, wl, preferred_element_type=jnp.float32)
          + jnp.dot(al, wh, preferred_element_type=jnp.float32))


def _spmm(l16, xhi, f):
  BM, BK = 2560, 1024
  nk = NP // BK

  def body(l_ref, x_ref, yhi_ref, ylo_ref, acc_ref):
    k = pl.program_id(1)

    @pl.when(k == 0)
    def _():
      acc_ref[...] = jnp.zeros_like(acc_ref)

    acc_ref[...] += jnp.dot(l_ref[...], x_ref[...],
                            preferred_element_type=jnp.float32)

    @pl.when(k == nk - 1)
    def _():
      hi, lo = _split(acc_ref[...])
      yhi_ref[...] = hi
      ylo_ref[...] = lo

  return pl.pallas_call(
      body,
      grid=(NP // BM, nk),
      in_specs=[
          pl.BlockSpec((BM, BK), lambda i, k: (i, k)),
          pl.BlockSpec((BK, f), lambda i, k: (k, 0)),
      ],
      out_specs=[pl.BlockSpec((BM, f), lambda i, k: (i, 0))] * 2,
      out_shape=[jax.ShapeDtypeStruct((NP, f), jnp.bfloat16)] * 2,
      scratch_shapes=[pltpu.VMEM((BM, f), jnp.float32)],
      compiler_params=pltpu.CompilerParams(
          dimension_semantics=("parallel", "arbitrary")),
  )(l16, xhi)


def _gate(ah, al, wp, wq, wr, pb, qb, rb):
  p = _mm3(ah, al, wp) + pb
  q = _mm3(ah, al, wq) + qb
  r = _mm3(ah, al, wr) + rb
  return jax.nn.relu(p * jax.nn.sigmoid(q) + r)


def _tc1_call(xnm, wp, wq, wr, pb, qb, rb):
  BM = 2560
  f_in, f_out = xnm.shape[1], pb.shape[1]

  def body(x_ref, wph, wpl, wqh, wql, wrh, wrl, pb_r, qb_r, rb_r,
           hhi_ref, hlo_ref):
    ah, al = _split(x_ref[...])
    h = _gate(ah, al, (wph[...], wpl[...]), (wqh[...], wql[...]),
              (wrh[...], wrl[...]), pb_r[...], qb_r[...], rb_r[...])
    hi, lo = _split(h)
    hhi_ref[...] = hi
    hlo_ref[...] = lo

  full = lambda shape: pl.BlockSpec(shape, lambda i: (0, 0))
  return pl.pallas_call(
      body,
      grid=(NP // BM,),
      in_specs=[pl.BlockSpec((BM, f_in), lambda i: (i, 0))]
      + [full((f_in, f_out))] * 6 + [full((1, f_out))] * 3,
      out_specs=[pl.BlockSpec((BM, f_out), lambda i: (i, 0))] * 2,
      out_shape=[jax.ShapeDtypeStruct((NP, f_out), jnp.bfloat16)] * 2,
  )(xnm, wp[0], wp[1], wq[0], wq[1], wr[0], wr[1], pb, qb, rb)


def _epilogue_call(yhi, ylo, t1hi, t1lo, h0hi, h0lo, cheb_w, tc2_w, bn_gb,
                   tail_w, f_sizes, out_f32):
  BM = 1280
  f, f2, f3 = f_sizes
  ca, cb, cc, cbias = cheb_w
  wp, wq, wr, pb, qb, rb = tc2_w
  g_col, b_col = bn_gb
  twp, twq, twr, tpb, tqb, trb = tail_w

  def body(yhi_r, ylo_r, t1hi_r, t1lo_r, h0hi_r, h0lo_r,
           cah, cal, cbh, cbl, cch, ccl, cbias_r,
           wph, wpl, wqh, wql, wrh, wrl, pb_r, qb_r, rb_r,
           g_r, b_r,
           twph, twpl, twqh, twql, twrh, twrl, tpb_r, tqb_r, trb_r,
           out_hi_ref, out_lo_ref):
    hc = jax.nn.relu(
        _mm3(h0hi_r[...], h0lo_r[...], (cah[...], cal[...]))
        + _mm3(t1hi_r[...], t1lo_r[...], (cbh[...], cbl[...]))
        + _mm3(yhi_r[...], ylo_r[...], (cch[...], ccl[...]))
        + cbias_r[...])
    hh, hl = _split(hc)
    gt = _gate(hh, hl, (wph[...], wpl[...]), (wqh[...], wql[...]),
               (wrh[...], wrl[...]), pb_r[...], qb_r[...], rb_r[...])
    mu = jnp.mean(gt, axis=1, keepdims=True)
    xc = gt - mu
    var = jnp.mean(xc * xc, axis=1, keepdims=True)
    xn = xc * lax.rsqrt(var + 1e-5) * g_r[...] + b_r[...]
    xh, xl = _split(xn)
    if out_f32:
      out_hi_ref[...] = (_mm3(xh, xl, (twph[...], twpl[...]))
                         + tpb_r[...])
      out_lo_ref[...] = jnp.zeros(out_lo_ref.shape, out_lo_ref.dtype)
    else:
      h2 = _gate(xh, xl, (twph[...], twpl[...]), (twqh[...], twql[...]),
                 (twrh[...], twrl[...]), tpb_r[...], tqb_r[...], trb_r[...])
      hi, lo = _split(h2)
      out_hi_ref[...] = hi
      out_lo_ref[...] = lo

  fo = tpb.shape[1]
  odt = jnp.float32 if out_f32 else jnp.bfloat16
  blk = lambda w: pl.BlockSpec(w.shape, lambda i: tuple(0 for _ in w.shape))
  row = lambda ff: pl.BlockSpec((BM, ff), lambda i: (i, 0))
  ins = [yhi, ylo, t1hi, t1lo, h0hi, h0lo,
         ca[0], ca[1], cb[0], cb[1], cc[0], cc[1], cbias,
         wp[0], wp[1], wq[0], wq[1], wr[0], wr[1], pb, qb, rb,
         g_col, b_col,
         twp[0], twp[1], twq[0], twq[1], twr[0], twr[1], tpb, tqb, trb]
  in_specs = ([row(f)] * 6
              + [blk(a) for a in ins[6:22]]
              + [pl.BlockSpec((BM, 1), lambda i: (i, 0))] * 2
              + [blk(a) for a in ins[24:]])
  return pl.pallas_call(
      body,
      grid=(NP // BM,),
      in_specs=in_specs,
      out_specs=[pl.BlockSpec((BM, fo), lambda i: (i, 0))] * 2,
      out_shape=[jax.ShapeDtypeStruct((NP, fo), odt)] * 2,
  )(*ins)


def _tconv_mats(w, b, t_in, i_ch, o_ch):
  ks = w.shape[-1]
  t_out = t_in - ks + 1
  mats, biases = [], []
  for gi in range(3):
    m1 = sum(
        jnp.einsum("ab,io->aibo",
                   jnp.eye(t_in, t_out, -kk, dtype=jnp.float32),
                   w[gi, :, :, 0, kk].T)
        for kk in range(ks))
    m = jnp.einsum("xy,tiso->xtiyso", jnp.eye(B, dtype=jnp.float32), m1)
    mats.append(_split(m.reshape(B * t_in * i_ch, B * t_out * o_ch)))
    biases.append(jnp.broadcast_to(b[gi], (B, t_out, o_ch)).reshape(1, -1))
  return mats, biases


def _cheb_mats(chw, chb, bt):
  eye = jnp.eye(bt, dtype=jnp.float32)
  ca = _split(jnp.kron(eye, (chw[0] - chw[2]).T))
  cb = _split(jnp.kron(eye, chw[1].T))
  cc = _split(jnp.kron(eye, 2.0 * chw[2].T))
  cbias = jnp.broadcast_to(chb, (bt, chw.shape[1])).reshape(1, -1)
  return ca, cb, cc, cbias


def _pad_rows(a, n_to):
  return jnp.pad(a, ((0, n_to - a.shape[0]),) + ((0, 0),) * (a.ndim - 1))


def kernel(x, edge_index, edge_weight, s1_tc1_w, s1_tc1_b, s1_cheb_w,
           s1_cheb_b, s1_tc2_w, s1_tc2_b, s1_bn_g, s1_bn_b, s2_tc1_w,
           s2_tc1_b, s2_cheb_w, s2_cheb_b, s2_tc2_w, s2_tc2_b, s2_bn_g,
           s2_bn_b, w_final, b_final, w_lin, b_lin):
  row = jnp.pad(edge_index[0].astype(jnp.int32),
                (0, EP - E)).reshape(ROWS_E, CHUNK)
  col = jnp.pad(edge_index[1].astype(jnp.int32),
                (0, EP - E)).reshape(ROWS_E, CHUNK)
  ew = jnp.pad(edge_weight, (0, EP - E)).reshape(ROWS_E, CHUNK)

  d_tab, degp = _sc_deg_and_owner(row, col, ew)
  dinv = _tc_dinv(degp).reshape(NP)
  win, lwp = _sc_lw_sums(row, col, ew, dinv, d_tab)
  keys, words = _tc_pack(row, col, win, lwp)
  l_words = _sc_build_l(keys, words)
  l16 = lax.bitcast_convert_type(
      l_words.reshape(NP, NW), jnp.bfloat16).reshape(NP, NP)

  tc1m, tc1b = _tconv_mats(s1_tc1_w, s1_tc1_b, T, 1, 32)
  cheb1 = _cheb_mats(s1_cheb_w, s1_cheb_b, B * 10)
  tc2m1, tc2b1 = _tconv_mats(s1_tc2_w, s1_tc2_b, 10, 32, 32)
  tc1m2, tc1b2 = _tconv_mats(s2_tc1_w, s2_tc1_b, 8, 32, 32)
  cheb2 = _cheb_mats(s2_cheb_w, s2_cheb_b, B * 6)
  tc2m2, tc2b2 = _tconv_mats(s2_tc2_w, s2_tc2_b, 6, 32, 32)
  wh_mat = jnp.kron(jnp.eye(B * 4, dtype=jnp.float32),
                    w_final @ w_lin.T)
  bh = jnp.broadcast_to(b_final @ w_lin.T + b_lin, (B * 4, 2)).reshape(1, -1)
  wh_pair = _split(wh_mat)
  g1 = _pad_rows(s1_bn_g.reshape(N, 1), NP)
  b1 = _pad_rows(s1_bn_b.reshape(N, 1), NP)
  g2 = _pad_rows(s2_bn_g.reshape(N, 1), NP)
  b2 = _pad_rows(s2_bn_b.reshape(N, 1), NP)

  xnm = _pad_rows(jnp.transpose(x[:, :, :, 0], (2, 0, 1)).reshape(N, B * T),
                  NP)
  h1hi, h1lo = _tc1_call(xnm, tc1m[0], tc1m[1], tc1m[2], *tc1b)
  t1hi, t1lo = _spmm(l16, h1hi, 640)
  yhi, ylo = _spmm(l16, t1hi, 640)
  h2hi, h2lo = _epilogue_call(
      yhi, ylo, t1hi, t1lo, h1hi, h1lo, cheb1,
      (tc2m1[0], tc2m1[1], tc2m1[2], *tc2b1), (g1, b1),
      (tc1m2[0], tc1m2[1], tc1m2[2], *tc1b2), (640, 512, 384), False)
  t2hi, t2lo = _spmm(l16, h2hi, 384)
  y2hi, y2lo = _spmm(l16, t2hi, 384)
  dummy = (jnp.zeros((1, 16), jnp.float32),) * 2
  ohi, _ = _epilogue_call(
      y2hi, y2lo, t2hi, t2lo, h2hi, h2lo, cheb2,
      (tc2m2[0], tc2m2[1], tc2m2[2], *tc2b2), (g2, b2),
      (wh_pair, (dummy[0].astype(jnp.bfloat16),) * 2,
       (dummy[0].astype(jnp.bfloat16),) * 2, bh, bh, bh),
      (384, 256, 16), True)
  return ohi[:N].reshape(N, B, 4, 2).transpose(1, 2, 0, 3)
